# baseline (speedup 1.0000x reference)
import functools

import jax
import jax.numpy as jnp
from jax import lax
from jax.experimental import pallas as pl
from jax.experimental.pallas import tpu as pltpu
from jax.experimental.pallas import tpu_sc as plsc

_B, _C, _H, _W = 8, 19, 512, 512
_NROWS = _B * _H
_NFLAT = _NROWS * _W
_NBLK = 32
_RB = _NROWS // _NBLK
_CHUNK = _RB * _W
_SC_TILE = 8192
_BUFN = _CHUNK + 128
_WINN = _SC_TILE + 32


def _stage1_body(pred_ref, sem_ref, pv_ref, slot_ref, stats_ref):
    neg_inf = jnp.float32(-jnp.inf)
    m1 = jnp.full((_RB, _W), neg_inf, jnp.float32)
    m2 = jnp.full((_RB, _W), neg_inf, jnp.float32)
    m3 = jnp.full((_RB, _W), neg_inf, jnp.float32)
    m4 = jnp.full((_RB, _W), neg_inf, jnp.float32)
    am = jnp.zeros((_RB, _W), jnp.int32)
    for k in range(_C):
        v = pred_ref[0, k]
        am = jnp.where(v > m1, k, am)
        t = jnp.maximum(m1, v)
        r = jnp.minimum(m1, v)
        m1 = t
        t = jnp.maximum(m2, r)
        r = jnp.minimum(m2, r)
        m2 = t
        t = jnp.maximum(m3, r)
        r = jnp.minimum(m3, r)
        m3 = t
        m4 = jnp.maximum(m4, r)
    e2 = jnp.exp(m2 - m1)
    e3 = jnp.exp(m3 - m1)
    e4 = jnp.exp(m4 - m1)
    rs = 1.0 / (1.0 + e2 + e3 + e4)

    def sig(x):
        return 1.0 / (1.0 + jnp.exp(-x))

    q = jnp.float32(0.25)
    s1 = sig(rs - q)
    s2 = sig(e2 * rs - q)
    s3 = sig(e3 * rs - q)
    s4 = sig(e4 * rs - q)
    mean = (s1 + s2 + s3 + s4) * jnp.float32(0.25)
    var = ((s1 - mean) ** 2 + (s2 - mean) ** 2 + (s3 - mean) ** 2
           + (s4 - mean) ** 2) * jnp.float32(1.0 / 3.0)
    pv_ref[...] = var
    mf = (am == sem_ref[0].astype(jnp.int32)).astype(jnp.float32)
    i0 = lax.broadcasted_iota(jnp.int32, (_W, _W), 0)
    i1 = lax.broadcasted_iota(jnp.int32, (_W, _W), 1)
    slt = (i0 < i1).astype(jnp.float32)
    lane_excl = jnp.dot(mf, slt, preferred_element_type=jnp.float32)
    rowtot = jnp.sum(mf, axis=1, keepdims=True)
    r0 = lax.broadcasted_iota(jnp.int32, (_RB, _RB), 0)
    r1 = lax.broadcasted_iota(jnp.int32, (_RB, _RB), 1)
    below = (r1 < r0).astype(jnp.float32)
    rowpref = jnp.sum(below * rowtot.reshape(1, _RB), axis=1, keepdims=True)
    r_excl = lane_excl + rowpref
    cnt = jnp.sum(mf)
    rowidx = lax.broadcasted_iota(jnp.int32, (_RB, _W), 0)
    lanef = lax.broadcasted_iota(jnp.int32, (_RB, _W), 1)
    posc = (rowidx * _W + lanef).astype(jnp.float32)
    slot = jnp.where(mf == 1.0, r_excl, cnt + (posc - r_excl))
    slot_ref[...] = slot.astype(jnp.int32)
    lane = lax.broadcasted_iota(jnp.int32, (1, 1, 128), 2)
    bmin = jnp.min(var)
    bmax = jnp.max(var)
    row = jnp.where(lane == 0, bmin,
                    jnp.where(lane == 1, bmax,
                              jnp.where(lane == 2, cnt, jnp.float32(0.0))))
    stats_ref[...] = row


def _z(i):
    return (i * 0).astype(jnp.int32) if hasattr(i, "astype") else 0


def _stage1(pred, sem):
    hb = _H // _RB
    return pl.pallas_call(
        _stage1_body,
        grid=(_NBLK,),
        in_specs=[
            pl.BlockSpec((1, _C, _RB, _W),
                         lambda i: (i // hb, _z(i), i % hb, _z(i))),
            pl.BlockSpec((1, _RB, _W), lambda i: (i // hb, i % hb, _z(i))),
        ],
        out_specs=[
            pl.BlockSpec((_RB, _W), lambda i: (i, _z(i))),
            pl.BlockSpec((_RB, _W), lambda i: (i, _z(i))),
            pl.BlockSpec((1, 1, 128), lambda i: (i, _z(i), _z(i))),
        ],
        out_shape=[
            jax.ShapeDtypeStruct((_NROWS + _RB, _W), jnp.float32),
            jax.ShapeDtypeStruct((_NROWS, _W), jnp.int32),
            jax.ShapeDtypeStruct((_NBLK, 1, 128), jnp.float32),
        ],
    )(pred, sem)


@functools.cache
def _make_stage2():
    mesh = plsc.VectorSubcoreMesh(core_axis_name="c", subcore_axis_name="s")
    rows1 = 8
    n1 = _RB // rows1
    win_rows = 32
    n2 = _CHUNK // _SC_TILE

    @functools.partial(
        pl.kernel,
        mesh=mesh,
        out_type=jax.ShapeDtypeStruct((_NBLK, 16), jnp.float32),
        compiler_params=pltpu.CompilerParams(needs_layout_passes=False),
        scratch_types=[
            pltpu.VMEM((2, rows1, _W), jnp.float32),
            pltpu.VMEM((2, rows1, _W), jnp.int32),
            pltpu.VMEM((_BUFN,), jnp.float32),
            pltpu.VMEM((2, win_rows, _W), jnp.float32),
            pltpu.VMEM((16,), jnp.int32),
            pltpu.VMEM((16,), jnp.float32),
            pltpu.VMEM((16,), jnp.float32),
            pltpu.VMEM((16,), jnp.float32),
            pltpu.SemaphoreType.DMA((2,)),
            pltpu.SemaphoreType.DMA((2,)),
            pltpu.SemaphoreType.DMA((2,)),
        ],
    )
    def stage2(pv_hbm, conf_hbm, slot_hbm, bases_hbm, vmin_hbm, inv_hbm,
               out_hbm, conf_t, slot_t, buf, win, bases_v, vmin_v, inv_v,
               out_v, csem, ssem, wsem):
        wid = lax.axis_index("s") * 2 + lax.axis_index("c")
        base_row = wid * _RB

        pltpu.sync_copy(bases_hbm.at[wid], bases_v)
        pltpu.sync_copy(vmin_hbm, vmin_v)
        pltpu.sync_copy(inv_hbm, inv_v)
        bv = bases_v[...]
        b1 = bv[0]
        cm = bv[1]
        b0 = wid * _CHUNK - b1
        c0 = _CHUNK - cm
        vminv = vmin_v[...]
        invv = inv_v[...]
        iota = lax.iota(jnp.int32, 16)

        def issue1(t):
            s = t % 2
            row = pl.multiple_of(base_row + t * rows1, 8)
            return (pltpu.async_copy(conf_hbm.at[pl.ds(row, rows1)],
                                     conf_t.at[jnp.int32(s)], csem.at[jnp.int32(s)]),
                    pltpu.async_copy(slot_hbm.at[pl.ds(row, rows1)],
                                     slot_t.at[jnp.int32(s)], ssem.at[jnp.int32(s)]))

        pend = issue1(0)
        for t in range(n1):
            nxt = issue1(t + 1) if t + 1 < n1 else None
            pend[0].wait()
            pend[1].wait()
            s = t % 2

            def row_loop(r, carry, s=s):
                def vec(j, carry):
                    for u in range(8):
                        c = conf_t[s, r, pl.ds(j * 128 + u * 16, 16)]
                        sl = slot_t[s, r, pl.ds(j * 128 + u * 16, 16)]
                        plsc.store_scatter(buf, [sl], c)
                    return carry

                return lax.fori_loop(jnp.int32(0), jnp.int32(_W // 128), vec,
                                     carry)

            lax.fori_loop(jnp.int32(0), jnp.int32(rows1), row_loop,
                          jnp.int32(0))
            pend = nxt

        max_row0 = (_NROWS + _RB) - win_rows

        def win_row0(stream_base, t):
            g = stream_base + t * _SC_TILE
            r = jnp.bitwise_and(lax.shift_right_logical(g, jnp.int32(9)),
                                jnp.int32(~7))
            return pl.multiple_of(jnp.minimum(r, max_row0), 8)

        def sweep(stream_base, cnt, sign, boff, acc):
            def issue2(t):
                s = t % 2
                return pltpu.async_copy(
                    pv_hbm.at[pl.ds(win_row0(stream_base, t), win_rows)],
                    win.at[jnp.int32(s)], wsem.at[jnp.int32(s)])

            pend = issue2(0)
            for t in range(n2):
                nxt = issue2(t + 1) if t + 1 < n2 else None
                pend.wait()
                s = t % 2
                start = t * _SC_TILE
                ln = jnp.clip(cnt - start, 0, _SC_TILE)
                sh = stream_base + start - win_row0(stream_base, t) * _W
                nv8 = (ln + 127) // 128

                def vec(i, acc, start=start, sh=sh, s=s):
                    for u in range(8):
                        rv = start + i * 128 + u * 16 + iota
                        valid = rv < cnt
                        a = plsc.load_gather(buf, [boff + rv])
                        q = sh + i * 128 + u * 16 + iota
                        v = plsc.load_gather(
                            win.at[jnp.int32(s)],
                            [lax.shift_right_logical(q, jnp.int32(9)),
                             jnp.bitwise_and(q, jnp.int32(511))])
                        pvn = (v - vminv) * invv
                        d = jnp.abs(pvn - jnp.float32(0.5))
                        outv = jnp.float32(0.5) + jnp.float32(sign) * d
                        term = jnp.abs(a - outv)
                        acc = acc + jnp.where(valid, term, jnp.float32(0.0))
                    return acc

                acc = lax.fori_loop(jnp.int32(0), nv8, vec, acc)
                pend = nxt
            return acc

        acc = jnp.zeros((16,), jnp.float32)
        acc = sweep(b1, cm, -1.0, jnp.int32(0), acc)
        acc = sweep(b0, c0, 1.0, cm, acc)
        out_v[...] = acc
        pltpu.sync_copy(out_v, out_hbm.at[wid])

    return stage2


def kernel(confidence, pred, sem_gt):
    sem = sem_gt.astype(jnp.uint32)
    pred = pred.astype(jnp.float32)
    conf = confidence.astype(jnp.float32)
    pv_pad, slot, stats = _stage1(pred, sem)
    st = stats[:, 0, :]
    vmin = jnp.min(st[:, 0])
    vmax = jnp.max(st[:, 1])
    cnts = st[:, 2].astype(jnp.int32)
    b1 = jnp.concatenate(
        [jnp.zeros((1,), jnp.int32), jnp.cumsum(cnts)[:-1].astype(jnp.int32)])
    bases = (jnp.zeros((_NBLK, 16), jnp.int32)
             .at[:, 0].set(b1).at[:, 1].set(cnts))
    inv = (1.0 / (vmax - vmin)).astype(jnp.float32)
    vminv = jnp.full((16,), vmin, jnp.float32)
    invv = jnp.full((16,), inv, jnp.float32)
    partials = _make_stage2()(pv_pad, conf.reshape(_NROWS, _W),
                              slot, bases, vminv, invv)
    return jnp.sum(partials) / jnp.float32(_NFLAT)


# speedup vs baseline: 1.0394x; 1.0394x over previous
import functools

import jax
import jax.numpy as jnp
from jax import lax
from jax.experimental import pallas as pl
from jax.experimental.pallas import tpu as pltpu
from jax.experimental.pallas import tpu_sc as plsc

_B, _C, _H, _W = 8, 19, 512, 512
_NROWS = _B * _H
_NFLAT = _NROWS * _W
_NBLK = 32
_RB = _NROWS // _NBLK
_CHUNK = _RB * _W
_SC_TILE = 8192
_BUFN = _CHUNK + 128
_WINN = _SC_TILE + 32


def _stage1_body(pred_ref, sem_ref, pv_ref, slot_ref, stats_ref):
    neg_inf = jnp.float32(-jnp.inf)
    m1 = jnp.full((_RB, _W), neg_inf, jnp.float32)
    m2 = jnp.full((_RB, _W), neg_inf, jnp.float32)
    m3 = jnp.full((_RB, _W), neg_inf, jnp.float32)
    m4 = jnp.full((_RB, _W), neg_inf, jnp.float32)
    am = jnp.zeros((_RB, _W), jnp.int32)
    for k in range(_C):
        v = pred_ref[0, k]
        am = jnp.where(v > m1, k, am)
        t = jnp.maximum(m1, v)
        r = jnp.minimum(m1, v)
        m1 = t
        t = jnp.maximum(m2, r)
        r = jnp.minimum(m2, r)
        m2 = t
        t = jnp.maximum(m3, r)
        r = jnp.minimum(m3, r)
        m3 = t
        m4 = jnp.maximum(m4, r)
    e2 = jnp.exp(m2 - m1)
    e3 = jnp.exp(m3 - m1)
    e4 = jnp.exp(m4 - m1)
    rs = 1.0 / (1.0 + e2 + e3 + e4)

    def sig(x):
        return 1.0 / (1.0 + jnp.exp(-x))

    q = jnp.float32(0.25)
    s1 = sig(rs - q)
    s2 = sig(e2 * rs - q)
    s3 = sig(e3 * rs - q)
    s4 = sig(e4 * rs - q)
    mean = (s1 + s2 + s3 + s4) * jnp.float32(0.25)
    var = ((s1 - mean) ** 2 + (s2 - mean) ** 2 + (s3 - mean) ** 2
           + (s4 - mean) ** 2) * jnp.float32(1.0 / 3.0)
    pv_ref[...] = var
    mf = (am == sem_ref[0].astype(jnp.int32)).astype(jnp.float32)
    i0 = lax.broadcasted_iota(jnp.int32, (_W, _W), 0)
    i1 = lax.broadcasted_iota(jnp.int32, (_W, _W), 1)
    slt = (i0 < i1).astype(jnp.float32)
    lane_excl = jnp.dot(mf, slt, preferred_element_type=jnp.float32)
    rowtot = jnp.sum(mf, axis=1, keepdims=True)
    r0 = lax.broadcasted_iota(jnp.int32, (_RB, _RB), 0)
    r1 = lax.broadcasted_iota(jnp.int32, (_RB, _RB), 1)
    below = (r1 < r0).astype(jnp.float32)
    rowpref = jnp.sum(below * rowtot.reshape(1, _RB), axis=1, keepdims=True)
    r_excl = lane_excl + rowpref
    cnt = jnp.sum(mf)
    rowidx = lax.broadcasted_iota(jnp.int32, (_RB, _W), 0)
    lanef = lax.broadcasted_iota(jnp.int32, (_RB, _W), 1)
    posc = (rowidx * _W + lanef).astype(jnp.float32)
    slot = jnp.where(mf == 1.0, r_excl, cnt + (posc - r_excl))
    slot_ref[...] = slot.astype(jnp.int32)
    lane = lax.broadcasted_iota(jnp.int32, (1, 1, 128), 2)
    bmin = jnp.min(var)
    bmax = jnp.max(var)
    row = jnp.where(lane == 0, bmin,
                    jnp.where(lane == 1, bmax,
                              jnp.where(lane == 2, cnt, jnp.float32(0.0))))
    stats_ref[...] = row


def _z(i):
    return (i * 0).astype(jnp.int32) if hasattr(i, "astype") else 0


def _stage1(pred, sem):
    hb = _H // _RB
    return pl.pallas_call(
        _stage1_body,
        grid=(_NBLK,),
        in_specs=[
            pl.BlockSpec((1, _C, _RB, _W),
                         lambda i: (i // hb, _z(i), i % hb, _z(i))),
            pl.BlockSpec((1, _RB, _W), lambda i: (i // hb, i % hb, _z(i))),
        ],
        out_specs=[
            pl.BlockSpec((_RB, _W), lambda i: (i, _z(i))),
            pl.BlockSpec((_RB, _W), lambda i: (i, _z(i))),
            pl.BlockSpec((1, 1, 128), lambda i: (i, _z(i), _z(i))),
        ],
        out_shape=[
            jax.ShapeDtypeStruct((_NROWS + _RB, _W), jnp.float32),
            jax.ShapeDtypeStruct((_NROWS, _W), jnp.int32),
            jax.ShapeDtypeStruct((_NBLK, 1, 128), jnp.float32),
        ],
    )(pred, sem)


@functools.cache
def _make_stage2():
    mesh = plsc.VectorSubcoreMesh(core_axis_name="c", subcore_axis_name="s")
    rows1 = 8
    n1 = _RB // rows1
    win_rows = 24
    n2 = _CHUNK // _SC_TILE

    @functools.partial(
        pl.kernel,
        mesh=mesh,
        out_type=jax.ShapeDtypeStruct((_NBLK, 16), jnp.float32),
        compiler_params=pltpu.CompilerParams(needs_layout_passes=False),
        scratch_types=[
            pltpu.VMEM((2, rows1, _W), jnp.float32),
            pltpu.VMEM((2, rows1, _W), jnp.int32),
            pltpu.VMEM((_BUFN,), jnp.float32),
            pltpu.VMEM((2, win_rows, _W), jnp.float32),
            pltpu.VMEM((16,), jnp.int32),
            pltpu.VMEM((16,), jnp.float32),
            pltpu.VMEM((16,), jnp.float32),
            pltpu.VMEM((16,), jnp.float32),
            pltpu.SemaphoreType.DMA((2,)),
            pltpu.SemaphoreType.DMA((2,)),
            pltpu.SemaphoreType.DMA((2,)),
        ],
    )
    def stage2(pv_hbm, conf_hbm, slot_hbm, bases_hbm, vmin_hbm, inv_hbm,
               out_hbm, conf_t, slot_t, buf, win, bases_v, vmin_v, inv_v,
               out_v, csem, ssem, wsem):
        wid = lax.axis_index("s") * 2 + lax.axis_index("c")
        base_row = wid * _RB

        pltpu.sync_copy(bases_hbm.at[wid], bases_v)
        pltpu.sync_copy(vmin_hbm, vmin_v)
        pltpu.sync_copy(inv_hbm, inv_v)
        bv = bases_v[...]
        b1 = bv[0]
        cm = bv[1]
        b0 = wid * _CHUNK - b1
        c0 = _CHUNK - cm
        vminv = vmin_v[...]
        invv = inv_v[...]
        iota = lax.iota(jnp.int32, 16)

        def issue1(t):
            s = t % 2
            row = pl.multiple_of(base_row + t * rows1, 8)
            return (pltpu.async_copy(conf_hbm.at[pl.ds(row, rows1)],
                                     conf_t.at[jnp.int32(s)], csem.at[jnp.int32(s)]),
                    pltpu.async_copy(slot_hbm.at[pl.ds(row, rows1)],
                                     slot_t.at[jnp.int32(s)], ssem.at[jnp.int32(s)]))

        pend = issue1(0)
        for t in range(n1):
            nxt = issue1(t + 1) if t + 1 < n1 else None
            pend[0].wait()
            pend[1].wait()
            s = t % 2

            def row_loop(r, carry, s=s):
                def vec(j, carry):
                    for u in range(4):
                        c = conf_t[s, r, pl.ds(j * 64 + u * 16, 16)]
                        sl = slot_t[s, r, pl.ds(j * 64 + u * 16, 16)]
                        plsc.store_scatter(buf, [sl], c)
                    return carry

                return lax.fori_loop(jnp.int32(0), jnp.int32(_W // 64), vec,
                                     carry)

            lax.fori_loop(jnp.int32(0), jnp.int32(rows1), row_loop,
                          jnp.int32(0))
            pend = nxt

        max_row0 = (_NROWS + _RB) - win_rows

        def win_row0(stream_base, t):
            g = stream_base + t * _SC_TILE
            r = jnp.bitwise_and(lax.shift_right_logical(g, jnp.int32(9)),
                                jnp.int32(~7))
            return pl.multiple_of(jnp.minimum(r, max_row0), 8)

        def sweep(stream_base, cnt, sign, boff, acc):
            na = (cnt + _SC_TILE - 1) // _SC_TILE

            def issue2(t):
                s = t % 2

                @pl.when(t < na)
                def _():
                    pltpu.async_copy(
                        pv_hbm.at[pl.ds(win_row0(stream_base, t), win_rows)],
                        win.at[jnp.int32(s)], wsem.at[jnp.int32(s)])

            def wait2(t):
                s = t % 2

                @pl.when(t < na)
                def _():
                    pltpu.make_async_copy(
                        pv_hbm.at[pl.ds(win_row0(stream_base, t), win_rows)],
                        win.at[jnp.int32(s)], wsem.at[jnp.int32(s)]).wait()

            issue2(0)
            for t in range(n2):
                if t + 1 < n2:
                    issue2(t + 1)
                wait2(t)
                s = t % 2
                start = t * _SC_TILE
                ln = jnp.clip(cnt - start, 0, _SC_TILE)
                sh = stream_base + start - win_row0(stream_base, t) * _W
                nv4 = (ln + 63) // 64

                def vec(i, acc, start=start, sh=sh, s=s):
                    for u in range(4):
                        rv = start + i * 64 + u * 16 + iota
                        valid = rv < cnt
                        a = plsc.load_gather(buf, [boff + rv])
                        q = sh + i * 64 + u * 16 + iota
                        v = plsc.load_gather(
                            win.at[jnp.int32(s)],
                            [lax.shift_right_logical(q, jnp.int32(9)),
                             jnp.bitwise_and(q, jnp.int32(511))])
                        pvn = (v - vminv) * invv
                        d = jnp.abs(pvn - jnp.float32(0.5))
                        outv = jnp.float32(0.5) + jnp.float32(sign) * d
                        term = jnp.abs(a - outv)
                        acc = acc + jnp.where(valid, term, jnp.float32(0.0))
                    return acc

                acc = lax.fori_loop(jnp.int32(0), nv4, vec, acc)
            return acc

        acc = jnp.zeros((16,), jnp.float32)
        acc = sweep(b1, cm, -1.0, jnp.int32(0), acc)
        acc = sweep(b0, c0, 1.0, cm, acc)
        out_v[...] = acc
        pltpu.sync_copy(out_v, out_hbm.at[wid])

    return stage2


def kernel(confidence, pred, sem_gt):
    sem = sem_gt.astype(jnp.uint32)
    pred = pred.astype(jnp.float32)
    conf = confidence.astype(jnp.float32)
    pv_pad, slot, stats = _stage1(pred, sem)
    st = stats[:, 0, :]
    vmin = jnp.min(st[:, 0])
    vmax = jnp.max(st[:, 1])
    cnts = st[:, 2].astype(jnp.int32)
    b1 = jnp.concatenate(
        [jnp.zeros((1,), jnp.int32), jnp.cumsum(cnts)[:-1].astype(jnp.int32)])
    bases = (jnp.zeros((_NBLK, 16), jnp.int32)
             .at[:, 0].set(b1).at[:, 1].set(cnts))
    inv = (1.0 / (vmax - vmin)).astype(jnp.float32)
    vminv = jnp.full((16,), vmin, jnp.float32)
    invv = jnp.full((16,), inv, jnp.float32)
    partials = _make_stage2()(pv_pad, conf.reshape(_NROWS, _W),
                              slot, bases, vminv, invv)
    return jnp.sum(partials) / jnp.float32(_NFLAT)


# speedup vs baseline: 1.0729x; 1.0322x over previous
import functools

import jax
import jax.numpy as jnp
from jax import lax
from jax.experimental import pallas as pl
from jax.experimental.pallas import tpu as pltpu
from jax.experimental.pallas import tpu_sc as plsc

_B, _C, _H, _W = 8, 19, 512, 512
_NROWS = _B * _H
_NFLAT = _NROWS * _W
_NBLK = 32
_RB = _NROWS // _NBLK
_CHUNK = _RB * _W
_SC_TILE = 8192
_BUFN = _CHUNK + 128
_WINN = _SC_TILE + 32


def _stage1_body(pred_ref, sem_ref, pv_ref, slot_ref, bases_ref, params_ref,
                 smin_ref, smax_ref, scnt_ref):
    neg_inf = jnp.float32(-jnp.inf)
    m1 = jnp.full((_RB, _W), neg_inf, jnp.float32)
    m2 = jnp.full((_RB, _W), neg_inf, jnp.float32)
    m3 = jnp.full((_RB, _W), neg_inf, jnp.float32)
    m4 = jnp.full((_RB, _W), neg_inf, jnp.float32)
    am = jnp.zeros((_RB, _W), jnp.int32)
    for k in range(_C):
        v = pred_ref[0, k]
        am = jnp.where(v > m1, k, am)
        t = jnp.maximum(m1, v)
        r = jnp.minimum(m1, v)
        m1 = t
        t = jnp.maximum(m2, r)
        r = jnp.minimum(m2, r)
        m2 = t
        t = jnp.maximum(m3, r)
        r = jnp.minimum(m3, r)
        m3 = t
        m4 = jnp.maximum(m4, r)
    e2 = jnp.exp(m2 - m1)
    e3 = jnp.exp(m3 - m1)
    e4 = jnp.exp(m4 - m1)
    rs = 1.0 / (1.0 + e2 + e3 + e4)

    def sig(x):
        return 1.0 / (1.0 + jnp.exp(-x))

    q = jnp.float32(0.25)
    s1 = sig(rs - q)
    s2 = sig(e2 * rs - q)
    s3 = sig(e3 * rs - q)
    s4 = sig(e4 * rs - q)
    mean = (s1 + s2 + s3 + s4) * jnp.float32(0.25)
    var = ((s1 - mean) ** 2 + (s2 - mean) ** 2 + (s3 - mean) ** 2
           + (s4 - mean) ** 2) * jnp.float32(1.0 / 3.0)
    pv_ref[...] = var
    mf = (am == sem_ref[0].astype(jnp.int32)).astype(jnp.float32)
    i0 = lax.broadcasted_iota(jnp.int32, (_W, _W), 0)
    i1 = lax.broadcasted_iota(jnp.int32, (_W, _W), 1)
    slt = (i0 < i1).astype(jnp.float32)
    lane_excl = jnp.dot(mf, slt, preferred_element_type=jnp.float32)
    rowtot = jnp.sum(mf, axis=1, keepdims=True)
    r0 = lax.broadcasted_iota(jnp.int32, (_RB, _RB), 0)
    r1 = lax.broadcasted_iota(jnp.int32, (_RB, _RB), 1)
    below = (r1 < r0).astype(jnp.float32)
    rowpref = jnp.sum(below * rowtot.reshape(1, _RB), axis=1, keepdims=True)
    r_excl = lane_excl + rowpref
    cnt = jnp.sum(mf)
    rowidx = lax.broadcasted_iota(jnp.int32, (_RB, _W), 0)
    lanef = lax.broadcasted_iota(jnp.int32, (_RB, _W), 1)
    posc = (rowidx * _W + lanef).astype(jnp.float32)
    slot = jnp.where(mf == 1.0, r_excl, cnt + (posc - r_excl))
    slot_ref[...] = slot.astype(jnp.int32)
    i = pl.program_id(0)
    smin_ref[pl.ds(i, 1)] = jnp.full((1, 128), jnp.min(var), jnp.float32)
    smax_ref[pl.ds(i, 1)] = jnp.full((1, 128), jnp.max(var), jnp.float32)
    scnt_ref[pl.ds(i, 1)] = jnp.full((1, 128), cnt, jnp.float32)

    @pl.when(i == _NBLK - 1)
    def _finalize():
        vmin = jnp.min(smin_ref[...])
        vmax = jnp.max(smax_ref[...])
        inv = 1.0 / (vmax - vmin)
        w0 = lax.broadcasted_iota(jnp.int32, (_NBLK, _NBLK), 0)
        w1 = lax.broadcasted_iota(jnp.int32, (_NBLK, _NBLK), 1)
        tri = (w1 < w0).astype(jnp.float32)
        b1m = jnp.dot(tri, scnt_ref[...],
                      preferred_element_type=jnp.float32)
        lane16 = lax.broadcasted_iota(jnp.int32, (_NBLK, 16), 1)
        bases_ref[...] = jnp.where(
            lane16 == 0, b1m[:, :16],
            jnp.where(lane16 == 1, scnt_ref[:, :16],
                      jnp.float32(0.0))).astype(jnp.int32)
        prow = lax.broadcasted_iota(jnp.int32, (8, 16), 0)
        params_ref[...] = jnp.where(prow == 0, vmin,
                                    jnp.where(prow == 1, inv,
                                              jnp.float32(0.0)))


def _z(i):
    return (i * 0).astype(jnp.int32) if hasattr(i, "astype") else 0


def _stage1(pred, sem):
    hb = _H // _RB
    return pl.pallas_call(
        _stage1_body,
        grid=(_NBLK,),
        in_specs=[
            pl.BlockSpec((1, _C, _RB, _W),
                         lambda i: (i // hb, _z(i), i % hb, _z(i))),
            pl.BlockSpec((1, _RB, _W), lambda i: (i // hb, i % hb, _z(i))),
        ],
        out_specs=[
            pl.BlockSpec((_RB, _W), lambda i: (i, _z(i))),
            pl.BlockSpec((_RB, _W), lambda i: (i, _z(i))),
            pl.BlockSpec((_NBLK, 16), lambda i: (_z(i), _z(i))),
            pl.BlockSpec((8, 16), lambda i: (_z(i), _z(i))),
        ],
        out_shape=[
            jax.ShapeDtypeStruct((_NROWS + _RB, _W), jnp.float32),
            jax.ShapeDtypeStruct((_NROWS, _W), jnp.int32),
            jax.ShapeDtypeStruct((_NBLK, 16), jnp.int32),
            jax.ShapeDtypeStruct((8, 16), jnp.float32),
        ],
        scratch_shapes=[
            pltpu.VMEM((_NBLK, 128), jnp.float32),
            pltpu.VMEM((_NBLK, 128), jnp.float32),
            pltpu.VMEM((_NBLK, 128), jnp.float32),
        ],
    )(pred, sem)


@functools.cache
def _make_stage2():
    mesh = plsc.VectorSubcoreMesh(core_axis_name="c", subcore_axis_name="s")
    rows1 = 8
    n1 = _RB // rows1
    win_rows = 24
    n2 = _CHUNK // _SC_TILE

    @functools.partial(
        pl.kernel,
        mesh=mesh,
        out_type=jax.ShapeDtypeStruct((_NBLK, 16), jnp.float32),
        compiler_params=pltpu.CompilerParams(needs_layout_passes=False),
        scratch_types=[
            pltpu.VMEM((2, rows1, _W), jnp.float32),
            pltpu.VMEM((2, rows1, _W), jnp.int32),
            pltpu.VMEM((_BUFN,), jnp.float32),
            pltpu.VMEM((2, win_rows, _W), jnp.float32),
            pltpu.VMEM((16,), jnp.int32),
            pltpu.VMEM((16,), jnp.float32),
            pltpu.VMEM((16,), jnp.float32),
            pltpu.VMEM((16,), jnp.float32),
            pltpu.SemaphoreType.DMA((2,)),
            pltpu.SemaphoreType.DMA((2,)),
            pltpu.SemaphoreType.DMA((2,)),
        ],
    )
    def stage2(pv_hbm, conf_hbm, slot_hbm, bases_hbm, params_hbm,
               out_hbm, conf_t, slot_t, buf, win, bases_v, vmin_v, inv_v,
               out_v, csem, ssem, wsem):
        wid = lax.axis_index("s") * 2 + lax.axis_index("c")
        base_row = wid * _RB

        pltpu.sync_copy(bases_hbm.at[wid], bases_v)
        pltpu.sync_copy(params_hbm.at[jnp.int32(0)], vmin_v)
        pltpu.sync_copy(params_hbm.at[jnp.int32(1)], inv_v)
        bv = bases_v[...]
        b1 = bv[0]
        cm = bv[1]
        b0 = wid * _CHUNK - b1
        c0 = _CHUNK - cm
        vminv = vmin_v[...]
        invv = inv_v[...]
        iota = lax.iota(jnp.int32, 16)

        def issue1(t):
            s = t % 2
            row = pl.multiple_of(base_row + t * rows1, 8)
            return (pltpu.async_copy(conf_hbm.at[pl.ds(row, rows1)],
                                     conf_t.at[jnp.int32(s)], csem.at[jnp.int32(s)]),
                    pltpu.async_copy(slot_hbm.at[pl.ds(row, rows1)],
                                     slot_t.at[jnp.int32(s)], ssem.at[jnp.int32(s)]))

        pend = issue1(0)
        for t in range(n1):
            nxt = issue1(t + 1) if t + 1 < n1 else None
            pend[0].wait()
            pend[1].wait()
            s = t % 2

            def row_loop(r, carry, s=s):
                def vec(j, carry):
                    for u in range(4):
                        c = conf_t[s, r, pl.ds(j * 64 + u * 16, 16)]
                        sl = slot_t[s, r, pl.ds(j * 64 + u * 16, 16)]
                        plsc.store_scatter(buf, [sl], c)
                    return carry

                return lax.fori_loop(jnp.int32(0), jnp.int32(_W // 64), vec,
                                     carry)

            lax.fori_loop(jnp.int32(0), jnp.int32(rows1), row_loop,
                          jnp.int32(0))
            pend = nxt

        max_row0 = (_NROWS + _RB) - win_rows

        def win_row0(stream_base, t):
            g = stream_base + t * _SC_TILE
            r = jnp.bitwise_and(lax.shift_right_logical(g, jnp.int32(9)),
                                jnp.int32(~7))
            return pl.multiple_of(jnp.minimum(r, max_row0), 8)

        def sweep(stream_base, cnt, sign, boff, acc):
            na = (cnt + _SC_TILE - 1) // _SC_TILE

            def issue2(t):
                s = t % 2

                @pl.when(t < na)
                def _():
                    pltpu.async_copy(
                        pv_hbm.at[pl.ds(win_row0(stream_base, t), win_rows)],
                        win.at[jnp.int32(s)], wsem.at[jnp.int32(s)])

            def wait2(t):
                s = t % 2

                @pl.when(t < na)
                def _():
                    pltpu.make_async_copy(
                        pv_hbm.at[pl.ds(win_row0(stream_base, t), win_rows)],
                        win.at[jnp.int32(s)], wsem.at[jnp.int32(s)]).wait()

            issue2(0)
            for t in range(n2):
                if t + 1 < n2:
                    issue2(t + 1)
                wait2(t)
                s = t % 2
                start = t * _SC_TILE
                ln = jnp.clip(cnt - start, 0, _SC_TILE)
                sh = stream_base + start - win_row0(stream_base, t) * _W
                nv4 = (ln + 63) // 64

                def vec(i, acc, start=start, sh=sh, s=s):
                    for u in range(4):
                        rv = start + i * 64 + u * 16 + iota
                        valid = rv < cnt
                        a = plsc.load_gather(buf, [boff + rv])
                        q = sh + i * 64 + u * 16 + iota
                        v = plsc.load_gather(
                            win.at[jnp.int32(s)],
                            [lax.shift_right_logical(q, jnp.int32(9)),
                             jnp.bitwise_and(q, jnp.int32(511))])
                        pvn = (v - vminv) * invv
                        d = jnp.abs(pvn - jnp.float32(0.5))
                        outv = jnp.float32(0.5) + jnp.float32(sign) * d
                        term = jnp.abs(a - outv)
                        acc = acc + jnp.where(valid, term, jnp.float32(0.0))
                    return acc

                acc = lax.fori_loop(jnp.int32(0), nv4, vec, acc)
            return acc

        acc = jnp.zeros((16,), jnp.float32)
        acc = sweep(b1, cm, -1.0, jnp.int32(0), acc)
        acc = sweep(b0, c0, 1.0, cm, acc)
        out_v[...] = acc
        pltpu.sync_copy(out_v, out_hbm.at[wid])

    return stage2


def kernel(confidence, pred, sem_gt):
    sem = sem_gt.astype(jnp.uint32)
    pred = pred.astype(jnp.float32)
    conf = confidence.astype(jnp.float32)
    pv_pad, slot, bases, params = _stage1(pred, sem)
    partials = _make_stage2()(pv_pad, conf.reshape(_NROWS, _W),
                              slot, bases, params)
    return jnp.sum(partials) / jnp.float32(_NFLAT)


# speedup vs baseline: 1.0732x; 1.0003x over previous
import functools

import jax
import jax.numpy as jnp
from jax import lax
from jax.experimental import pallas as pl
from jax.experimental.pallas import tpu as pltpu
from jax.experimental.pallas import tpu_sc as plsc

_B, _C, _H, _W = 8, 19, 512, 512
_NROWS = _B * _H
_NFLAT = _NROWS * _W
_NBLK = 32
_RB = _NROWS // _NBLK
_CHUNK = _RB * _W
_SC_TILE = 8192
_BUFN = _CHUNK + 128
_WINN = _SC_TILE + 32


def _stage1_body(pred_ref, sem_ref, pv_ref, slot_ref, bases_ref, params_ref,
                 smin_ref, smax_ref, scnt_ref):
    neg_inf = jnp.float32(-jnp.inf)
    m1 = jnp.full((_RB, _W), neg_inf, jnp.float32)
    m2 = jnp.full((_RB, _W), neg_inf, jnp.float32)
    m3 = jnp.full((_RB, _W), neg_inf, jnp.float32)
    m4 = jnp.full((_RB, _W), neg_inf, jnp.float32)
    am = jnp.zeros((_RB, _W), jnp.int32)
    for k in range(_C):
        v = pred_ref[0, k]
        am = jnp.where(v > m1, k, am)
        t = jnp.maximum(m1, v)
        r = jnp.minimum(m1, v)
        m1 = t
        t = jnp.maximum(m2, r)
        r = jnp.minimum(m2, r)
        m2 = t
        t = jnp.maximum(m3, r)
        r = jnp.minimum(m3, r)
        m3 = t
        m4 = jnp.maximum(m4, r)
    e2 = jnp.exp(m2 - m1)
    e3 = jnp.exp(m3 - m1)
    e4 = jnp.exp(m4 - m1)
    rs = 1.0 / (1.0 + e2 + e3 + e4)

    def sig(x):
        return 1.0 / (1.0 + jnp.exp(-x))

    q = jnp.float32(0.25)
    s1 = sig(rs - q)
    s2 = sig(e2 * rs - q)
    s3 = sig(e3 * rs - q)
    s4 = sig(e4 * rs - q)
    mean = (s1 + s2 + s3 + s4) * jnp.float32(0.25)
    var = ((s1 - mean) ** 2 + (s2 - mean) ** 2 + (s3 - mean) ** 2
           + (s4 - mean) ** 2) * jnp.float32(1.0 / 3.0)
    pv_ref[...] = var
    mf = (am == sem_ref[0].astype(jnp.int32)).astype(jnp.float32)
    i0 = lax.broadcasted_iota(jnp.int32, (_W, _W), 0)
    i1 = lax.broadcasted_iota(jnp.int32, (_W, _W), 1)
    slt = (i0 < i1).astype(jnp.float32)
    lane_excl = jnp.dot(mf, slt, preferred_element_type=jnp.float32)
    rowtot = jnp.sum(mf, axis=1, keepdims=True)
    r0 = lax.broadcasted_iota(jnp.int32, (_RB, _RB), 0)
    r1 = lax.broadcasted_iota(jnp.int32, (_RB, _RB), 1)
    below = (r1 < r0).astype(jnp.float32)
    rowpref = jnp.sum(below * rowtot.reshape(1, _RB), axis=1, keepdims=True)
    r_excl = lane_excl + rowpref
    cnt = jnp.sum(mf)
    rowidx = lax.broadcasted_iota(jnp.int32, (_RB, _W), 0)
    lanef = lax.broadcasted_iota(jnp.int32, (_RB, _W), 1)
    posc = (rowidx * _W + lanef).astype(jnp.float32)
    slot = jnp.where(mf == 1.0, r_excl, cnt + (posc - r_excl))
    slot_ref[...] = slot.astype(jnp.int32)
    i = pl.program_id(0)
    smin_ref[pl.ds(i, 1)] = jnp.full((1, 128), jnp.min(var), jnp.float32)
    smax_ref[pl.ds(i, 1)] = jnp.full((1, 128), jnp.max(var), jnp.float32)
    scnt_ref[pl.ds(i, 1)] = jnp.full((1, 128), cnt, jnp.float32)

    @pl.when(i == _NBLK - 1)
    def _finalize():
        vmin = jnp.min(smin_ref[...])
        vmax = jnp.max(smax_ref[...])
        inv = 1.0 / (vmax - vmin)
        w0 = lax.broadcasted_iota(jnp.int32, (_NBLK, _NBLK), 0)
        w1 = lax.broadcasted_iota(jnp.int32, (_NBLK, _NBLK), 1)
        tri = (w1 < w0).astype(jnp.float32)
        b1m = jnp.round(jnp.dot(tri, scnt_ref[...],
                                preferred_element_type=jnp.float32,
                                precision=lax.Precision.HIGHEST))
        lane16 = lax.broadcasted_iota(jnp.int32, (_NBLK, 16), 1)
        bases_ref[...] = jnp.where(
            lane16 == 0, b1m[:, :16],
            jnp.where(lane16 == 1, scnt_ref[:, :16],
                      jnp.float32(0.0))).astype(jnp.int32)
        prow = lax.broadcasted_iota(jnp.int32, (8, 16), 0)
        params_ref[...] = jnp.where(prow == 0, vmin,
                                    jnp.where(prow == 1, inv,
                                              jnp.float32(0.0)))


def _z(i):
    return (i * 0).astype(jnp.int32) if hasattr(i, "astype") else 0


def _stage1(pred, sem):
    hb = _H // _RB
    return pl.pallas_call(
        _stage1_body,
        grid=(_NBLK,),
        in_specs=[
            pl.BlockSpec((1, _C, _RB, _W),
                         lambda i: (i // hb, _z(i), i % hb, _z(i))),
            pl.BlockSpec((1, _RB, _W), lambda i: (i // hb, i % hb, _z(i))),
        ],
        out_specs=[
            pl.BlockSpec((_RB, _W), lambda i: (i, _z(i))),
            pl.BlockSpec((_RB, _W), lambda i: (i, _z(i))),
            pl.BlockSpec((_NBLK, 16), lambda i: (_z(i), _z(i))),
            pl.BlockSpec((8, 16), lambda i: (_z(i), _z(i))),
        ],
        out_shape=[
            jax.ShapeDtypeStruct((_NROWS + _RB, _W), jnp.float32),
            jax.ShapeDtypeStruct((_NROWS, _W), jnp.int32),
            jax.ShapeDtypeStruct((_NBLK, 16), jnp.int32),
            jax.ShapeDtypeStruct((8, 16), jnp.float32),
        ],
        scratch_shapes=[
            pltpu.VMEM((_NBLK, 128), jnp.float32),
            pltpu.VMEM((_NBLK, 128), jnp.float32),
            pltpu.VMEM((_NBLK, 128), jnp.float32),
        ],
    )(pred, sem)


@functools.cache
def _make_stage2():
    mesh = plsc.VectorSubcoreMesh(core_axis_name="c", subcore_axis_name="s")
    rows1 = 8
    n1 = _RB // rows1
    win_rows = 24
    n2 = _CHUNK // _SC_TILE

    @functools.partial(
        pl.kernel,
        mesh=mesh,
        out_type=jax.ShapeDtypeStruct((_NBLK, 16), jnp.float32),
        compiler_params=pltpu.CompilerParams(needs_layout_passes=False),
        scratch_types=[
            pltpu.VMEM((2, rows1, _W), jnp.float32),
            pltpu.VMEM((2, rows1, _W), jnp.int32),
            pltpu.VMEM((_BUFN,), jnp.float32),
            pltpu.VMEM((2, win_rows, _W), jnp.float32),
            pltpu.VMEM((16,), jnp.int32),
            pltpu.VMEM((16,), jnp.float32),
            pltpu.VMEM((16,), jnp.float32),
            pltpu.VMEM((16,), jnp.float32),
            pltpu.SemaphoreType.DMA((2,)),
            pltpu.SemaphoreType.DMA((2,)),
            pltpu.SemaphoreType.DMA((2,)),
        ],
    )
    def stage2(pv_hbm, conf_hbm, slot_hbm, bases_hbm, params_hbm,
               out_hbm, conf_t, slot_t, buf, win, bases_v, vmin_v, inv_v,
               out_v, csem, ssem, wsem):
        wid = lax.axis_index("s") * 2 + lax.axis_index("c")
        base_row = wid * _RB

        pltpu.sync_copy(bases_hbm.at[wid], bases_v)
        pltpu.sync_copy(params_hbm.at[jnp.int32(0)], vmin_v)
        pltpu.sync_copy(params_hbm.at[jnp.int32(1)], inv_v)
        bv = bases_v[...]
        b1 = bv[0]
        cm = bv[1]
        b0 = wid * _CHUNK - b1
        c0 = _CHUNK - cm
        vminv = vmin_v[...]
        invv = inv_v[...]
        iota = lax.iota(jnp.int32, 16)

        def issue1(t):
            s = t % 2
            row = pl.multiple_of(base_row + t * rows1, 8)
            return (pltpu.async_copy(conf_hbm.at[pl.ds(row, rows1)],
                                     conf_t.at[jnp.int32(s)], csem.at[jnp.int32(s)]),
                    pltpu.async_copy(slot_hbm.at[pl.ds(row, rows1)],
                                     slot_t.at[jnp.int32(s)], ssem.at[jnp.int32(s)]))

        pend = issue1(0)
        for t in range(n1):
            nxt = issue1(t + 1) if t + 1 < n1 else None
            pend[0].wait()
            pend[1].wait()
            s = t % 2

            def row_loop(r, carry, s=s):
                def vec(j, carry):
                    for u in range(4):
                        c = conf_t[s, r, pl.ds(j * 64 + u * 16, 16)]
                        sl = slot_t[s, r, pl.ds(j * 64 + u * 16, 16)]
                        plsc.store_scatter(buf, [sl], c)
                    return carry

                return lax.fori_loop(jnp.int32(0), jnp.int32(_W // 64), vec,
                                     carry)

            lax.fori_loop(jnp.int32(0), jnp.int32(rows1), row_loop,
                          jnp.int32(0))
            pend = nxt

        max_row0 = (_NROWS + _RB) - win_rows

        def win_row0(stream_base, t):
            g = stream_base + t * _SC_TILE
            r = jnp.bitwise_and(lax.shift_right_logical(g, jnp.int32(9)),
                                jnp.int32(~7))
            return pl.multiple_of(jnp.minimum(r, max_row0), 8)

        def sweep(stream_base, cnt, sign, boff, acc):
            na = (cnt + _SC_TILE - 1) // _SC_TILE

            def issue2(t):
                s = t % 2

                @pl.when(t < na)
                def _():
                    pltpu.async_copy(
                        pv_hbm.at[pl.ds(win_row0(stream_base, t), win_rows)],
                        win.at[jnp.int32(s)], wsem.at[jnp.int32(s)])

            def wait2(t):
                s = t % 2

                @pl.when(t < na)
                def _():
                    pltpu.make_async_copy(
                        pv_hbm.at[pl.ds(win_row0(stream_base, t), win_rows)],
                        win.at[jnp.int32(s)], wsem.at[jnp.int32(s)]).wait()

            issue2(0)
            for t in range(n2):
                if t + 1 < n2:
                    issue2(t + 1)
                wait2(t)
                s = t % 2
                start = t * _SC_TILE
                ln = jnp.clip(cnt - start, 0, _SC_TILE)
                sh = stream_base + start - win_row0(stream_base, t) * _W
                nv4 = (ln + 63) // 64

                def vec(i, acc, start=start, sh=sh, s=s):
                    for u in range(4):
                        rv = start + i * 64 + u * 16 + iota
                        valid = rv < cnt
                        a = plsc.load_gather(buf, [boff + rv])
                        q = sh + i * 64 + u * 16 + iota
                        v = plsc.load_gather(
                            win.at[jnp.int32(s)],
                            [lax.shift_right_logical(q, jnp.int32(9)),
                             jnp.bitwise_and(q, jnp.int32(511))])
                        pvn = (v - vminv) * invv
                        d = jnp.abs(pvn - jnp.float32(0.5))
                        outv = jnp.float32(0.5) + jnp.float32(sign) * d
                        term = jnp.abs(a - outv)
                        acc = acc + jnp.where(valid, term, jnp.float32(0.0))
                    return acc

                acc = lax.fori_loop(jnp.int32(0), nv4, vec, acc)
            return acc

        acc = jnp.zeros((16,), jnp.float32)
        acc = sweep(b1, cm, -1.0, jnp.int32(0), acc)
        acc = sweep(b0, c0, 1.0, cm, acc)
        out_v[...] = acc
        pltpu.sync_copy(out_v, out_hbm.at[wid])

    return stage2


def kernel(confidence, pred, sem_gt):
    sem = sem_gt.astype(jnp.uint32)
    pred = pred.astype(jnp.float32)
    conf = confidence.astype(jnp.float32)
    pv_pad, slot, bases, params = _stage1(pred, sem)
    partials = _make_stage2()(pv_pad, conf.reshape(_NROWS, _W),
                              slot, bases, params)
    return jnp.sum(partials) / jnp.float32(_NFLAT)
